# baseline (speedup 1.0000x reference)
import functools

import jax
import jax.numpy as jnp
from jax.experimental import pallas as pl
from jax.experimental.pallas import tpu as pltpu

NUM_CODE = 1024
DIM_CODE = 256
TOK_BLOCK = 4608
SUB_BLOCK = 512


def _vq_body(nblocks, total, z_ref, zn_ref, cn_ref, cb_ref,
             zq_ref, idx_ref, loss_ref, res_ref):
    cb = cb_ref[...]
    cn = cn_ref[...]
    parts = []
    for s in range(TOK_BLOCK // SUB_BLOCK):
        sl = pl.ds(s * SUB_BLOCK, SUB_BLOCK)
        z = z_ref[sl, :]
        zn = zn_ref[sl, :]
        zb = (2.0 * z).astype(jnp.bfloat16)
        mm = jax.lax.dot_general(zb, cb, (((1,), (1,)), ((), ())),
                                 preferred_element_type=jnp.float32)
        d = (zn + cn) - mm
        dmin = jnp.min(d, axis=1, keepdims=True)
        iota = jax.lax.broadcasted_iota(jnp.int32, d.shape, 1)
        idx = jnp.min(jnp.where(d == dmin, iota, NUM_CODE), axis=1)
        idx = idx.astype(jnp.int32)
        onehot = (idx[:, None] == iota).astype(jnp.float32)
        zq = jnp.dot(onehot, cb, preferred_element_type=jnp.float32)
        r = z - zq
        zq_ref[sl, :] = zq
        idx_ref[0, 0, sl] = idx
        res_ref[sl, :] = r
        parts.append(jnp.sum(r * r))
    loss_ref[0, 0, :] = jnp.broadcast_to(sum(parts), (128,))


@jax.jit
def kernel(z, codebook):
    B, L, E = z.shape
    n_tok = B * L
    z_flat = z.reshape(n_tok, E)
    nb = n_tok // TOK_BLOCK

    znorm = jnp.sum(z ** 2, axis=2).reshape(n_tok, 1)
    cnorm = jnp.sum(codebook ** 2, axis=1)[None, :]

    zq, idx, loss, res = pl.pallas_call(
        functools.partial(_vq_body, nb, z.size),
        grid=(nb,),
        in_specs=[
            pl.BlockSpec((TOK_BLOCK, E), lambda i: (i, 0)),
            pl.BlockSpec((TOK_BLOCK, 1), lambda i: (i, 0)),
            pl.BlockSpec((1, NUM_CODE), lambda i: (0, 0)),
            pl.BlockSpec((NUM_CODE, E), lambda i: (0, 0)),
        ],
        out_specs=[
            pl.BlockSpec((TOK_BLOCK, E), lambda i: (i, 0)),
            pl.BlockSpec((1, 1, TOK_BLOCK), lambda i: (i, 0, 0)),
            pl.BlockSpec((1, 1, 128), lambda i: (i, 0, 0)),
            pl.BlockSpec((TOK_BLOCK, E), lambda i: (i, 0)),
        ],
        out_shape=[
            jax.ShapeDtypeStruct((n_tok, E), jnp.float32),
            jax.ShapeDtypeStruct((nb, 1, TOK_BLOCK), jnp.int32),
            jax.ShapeDtypeStruct((nb, 1, 128), jnp.float32),
            jax.ShapeDtypeStruct((n_tok, E), jnp.float32),
        ],
        compiler_params=pltpu.CompilerParams(
            dimension_semantics=("parallel",)),
    )(z_flat, znorm, cnorm, codebook)

    loss = jnp.sum(loss[:, 0, 0]) * (2.0 / z.size)
    return (zq.reshape(B, L, E), idx.reshape(B, L), loss,
            res.reshape(B, L, E))


# speedup vs baseline: 1.0187x; 1.0187x over previous
import functools

import jax
import jax.numpy as jnp
from jax.experimental import pallas as pl
from jax.experimental.pallas import tpu as pltpu

NUM_CODE = 1024
DIM_CODE = 256
TOK_BLOCK = 2048
SUB_BLOCK = 256


def _vq_body(nblocks, total, z_ref, zn_ref, cn_ref, cb_ref,
             zq_ref, idx_ref, loss_ref, res_ref):
    cb = cb_ref[...]
    cn = cn_ref[...]
    parts = []
    for s in range(TOK_BLOCK // SUB_BLOCK):
        sl = pl.ds(s * SUB_BLOCK, SUB_BLOCK)
        z = z_ref[sl, :]
        zn = zn_ref[sl, :]
        zb = (2.0 * z).astype(jnp.bfloat16)
        mm = jax.lax.dot_general(zb, cb, (((1,), (1,)), ((), ())),
                                 preferred_element_type=jnp.float32)
        d = (zn + cn) - mm
        dmin = jnp.min(d, axis=1, keepdims=True)
        iota = jax.lax.broadcasted_iota(jnp.int32, d.shape, 1)
        idx = jnp.min(jnp.where(d == dmin, iota, NUM_CODE), axis=1)
        idx = idx.astype(jnp.int32)
        onehot = (idx[:, None] == iota).astype(jnp.float32)
        zq = jnp.dot(onehot, cb, preferred_element_type=jnp.float32)
        r = z - zq
        zq_ref[sl, :] = zq
        idx_ref[0, 0, sl] = idx
        res_ref[sl, :] = r
        parts.append(jnp.sum(r * r))
    loss_ref[0, 0, :] = jnp.broadcast_to(sum(parts), (128,))


@jax.jit
def kernel(z, codebook):
    B, L, E = z.shape
    n_tok = B * L
    z_flat = z.reshape(n_tok, E)
    nb = n_tok // TOK_BLOCK

    znorm = jnp.sum(z ** 2, axis=2).reshape(n_tok, 1)
    cnorm = jnp.sum(codebook ** 2, axis=1)[None, :]

    zq, idx, loss, res = pl.pallas_call(
        functools.partial(_vq_body, nb, z.size),
        grid=(nb,),
        in_specs=[
            pl.BlockSpec((TOK_BLOCK, E), lambda i: (i, 0)),
            pl.BlockSpec((TOK_BLOCK, 1), lambda i: (i, 0)),
            pl.BlockSpec((1, NUM_CODE), lambda i: (0, 0)),
            pl.BlockSpec((NUM_CODE, E), lambda i: (0, 0)),
        ],
        out_specs=[
            pl.BlockSpec((TOK_BLOCK, E), lambda i: (i, 0)),
            pl.BlockSpec((1, 1, TOK_BLOCK), lambda i: (i, 0, 0)),
            pl.BlockSpec((1, 1, 128), lambda i: (i, 0, 0)),
            pl.BlockSpec((TOK_BLOCK, E), lambda i: (i, 0)),
        ],
        out_shape=[
            jax.ShapeDtypeStruct((n_tok, E), jnp.float32),
            jax.ShapeDtypeStruct((nb, 1, TOK_BLOCK), jnp.int32),
            jax.ShapeDtypeStruct((nb, 1, 128), jnp.float32),
            jax.ShapeDtypeStruct((n_tok, E), jnp.float32),
        ],
        compiler_params=pltpu.CompilerParams(
            dimension_semantics=("parallel",)),
    )(z_flat, znorm, cnorm, codebook)

    loss = jnp.sum(loss[:, 0, 0]) * (2.0 / z.size)
    return (zq.reshape(B, L, E), idx.reshape(B, L), loss,
            res.reshape(B, L, E))
